# baseline (speedup 1.0000x reference)
import functools

import jax
import jax.numpy as jnp
from jax import lax
from jax.experimental import pallas as pl
from jax.experimental.pallas import tpu as pltpu
from jax.experimental.pallas import tpu_sc as plsc

NC, NS, L = 2, 16, 16
NW = NC * NS
B = 2
C = 128
NCLS = 2
N = 32 * 32 * 32
PPW = N // NW
WBLK = 128
NBLK = PPW // WBLK
NG = WBLK // L


def _masked_sqrt(x):
    xc = jnp.maximum(x, jnp.float32(1e-12))
    i = lax.bitcast_convert_type(xc, jnp.int32)
    seed = jnp.full((L,), 0x5F3759DF, jnp.int32) - (i >> 1)
    y = lax.bitcast_convert_type(seed, jnp.float32)
    for _ in range(3):
        y = y * (jnp.float32(1.5) - jnp.float32(0.5) * xc * y * y)
    return x * y


def _sc_body(feat, tgt, p0, p1, out, fblk, tgtv, p0v, p1v, outv):
    wid = lax.axis_index("s") * NC + lax.axis_index("c")
    base = wid * PPW
    pltpu.sync_copy(p0, p0v)
    pltpu.sync_copy(p1, p1v)
    zero = jnp.zeros((L,), jnp.float32)
    one = jnp.ones((L,), jnp.float32)
    accs_out = []
    for b in range(B):
        pltpu.sync_copy(tgt.at[b, pl.ds(base, PPW)], tgtv)
        sum0 = zero
        cnt0 = zero
        sum1 = zero
        cnt1 = zero
        for blk in range(NBLK):
            off = base + blk * WBLK
            pltpu.sync_copy(feat.at[b, :, pl.ds(off, WBLK)], fblk)

            def kbody(k, carry):
                pk0 = p0v[k, :]
                pk1 = p1v[k, :]
                new = []
                for g in range(NG):
                    v = fblk[k, pl.ds(g * L, L)]
                    d0 = v - pk0
                    d1 = v - pk1
                    new.append(carry[2 * g] + d0 * d0)
                    new.append(carry[2 * g + 1] + d1 * d1)
                return tuple(new)

            accs = lax.fori_loop(0, C, kbody, tuple(zero for _ in range(2 * NG)))
            for g in range(NG):
                t = tgtv[pl.ds(blk * WBLK + g * L, L)]
                d0 = _masked_sqrt(accs[2 * g])
                d1 = _masked_sqrt(accs[2 * g + 1])
                m0 = t == 0
                m1 = t == 1
                sum0 = sum0 + jnp.where(m0, d0, zero)
                cnt0 = cnt0 + jnp.where(m0, one, zero)
                sum1 = sum1 + jnp.where(m1, d1, zero)
                cnt1 = cnt1 + jnp.where(m1, one, zero)
        accs_out += [sum0, cnt0, sum1, cnt1]
    for j, v in enumerate(accs_out):
        outv[j, :] = v
    pltpu.sync_copy(outv, out.at[wid])


@functools.partial(jax.jit)
def _sc_call(feat, tgt, p0, p1):
    mesh = plsc.VectorSubcoreMesh(core_axis_name="c", subcore_axis_name="s")
    return pl.kernel(
        _sc_body,
        out_type=jax.ShapeDtypeStruct((NW, 2 * NCLS * B, L), jnp.float32),
        mesh=mesh,
        scratch_types=[
            pltpu.VMEM((C, WBLK), jnp.float32),
            pltpu.VMEM((PPW,), jnp.int32),
            pltpu.VMEM((C, L), jnp.float32),
            pltpu.VMEM((C, L), jnp.float32),
            pltpu.VMEM((2 * NCLS * B, L), jnp.float32),
        ],
    )(feat, tgt, p0, p1)


def kernel(features, predictions, targets, prototypes):
    del predictions
    feat = features.reshape(B, C, N)
    tgt = targets.reshape(B, N)
    p0 = jnp.broadcast_to(prototypes[0][:, None], (C, L))
    p1 = jnp.broadcast_to(prototypes[1][:, None], (C, L))
    part = _sc_call(feat, tgt, p0, p1)
    part = part.reshape(NW, B, NCLS, 2, L)
    sums = part.sum(axis=(0, 4))
    s = sums[..., 0]
    n = sums[..., 1]
    mean = jnp.where(n > 0, s / jnp.maximum(n, 1.0), 0.0)
    total = mean.sum()
    valid = (n > 0).astype(jnp.float32).sum()
    return jnp.where(valid > 0, total / valid, jnp.float32(0.0))


# speedup vs baseline: 1.2386x; 1.2386x over previous
import functools

import jax
import jax.numpy as jnp
from jax import lax
from jax.experimental import pallas as pl
from jax.experimental.pallas import tpu as pltpu
from jax.experimental.pallas import tpu_sc as plsc

NC, NS, L = 2, 16, 16
NW = NC * NS
B = 2
C = 128
NCLS = 2
N = 32 * 32 * 32
PPW = N // NW
WBLK = 128
NBLK = PPW // WBLK
NSTEP = B * NBLK
NG = WBLK // L
KU = 4


def _masked_sqrt(x):
    xc = jnp.maximum(x, jnp.float32(1e-12))
    i = lax.bitcast_convert_type(xc, jnp.int32)
    seed = jnp.full((L,), 0x5F3759DF, jnp.int32) - (i >> 1)
    y = lax.bitcast_convert_type(seed, jnp.float32)
    for _ in range(3):
        y = y * (jnp.float32(1.5) - jnp.float32(0.5) * xc * y * y)
    return x * y


def _sc_body(feat, tgt, p0, p1, out, fb0, fb1, tgtv, p0v, p1v, outv, sem0, sem1):
    wid = lax.axis_index("s") * NC + lax.axis_index("c")
    base = wid * PPW
    pltpu.sync_copy(p0, p0v)
    pltpu.sync_copy(p1, p1v)
    for b in range(B):
        pltpu.sync_copy(tgt.at[b, pl.ds(base, PPW)], tgtv.at[b])
    zero = jnp.zeros((L,), jnp.float32)
    one = jnp.ones((L,), jnp.float32)

    bufs = (fb0, fb1)
    sems = (sem0, sem1)

    def start(step):
        b, blk = divmod(step, NBLK)
        i = step % 2
        return pltpu.async_copy(
            feat.at[b, :, pl.ds(base + blk * WBLK, WBLK)], bufs[i], sems[i])

    copies = {0: start(0)}
    acc_bc = [[zero, zero, zero, zero] for _ in range(B)]
    for step in range(NSTEP):
        b, blk = divmod(step, NBLK)
        if step + 1 < NSTEP:
            copies[step + 1] = start(step + 1)
        copies.pop(step).wait()
        fblk = bufs[step % 2]

        masks = []
        for g in range(NG):
            t = tgtv[b, pl.ds(blk * WBLK + g * L, L)]
            masks.append((t == 0, t == 1))

        def kbody(i, carry):
            accs = list(carry)
            for kk in range(KU):
                k = i * KU + kk
                pk0 = p0v[k, :]
                pk1 = p1v[k, :]
                for g in range(NG):
                    v = fblk[k, pl.ds(g * L, L)]
                    psel = jnp.where(masks[g][0], pk0, pk1)
                    d = v - psel
                    accs[g] = accs[g] + d * d
            return tuple(accs)

        accs = lax.fori_loop(0, C // KU, kbody, tuple(zero for _ in range(NG)))
        s0, c0, s1, c1 = acc_bc[b]
        for g in range(NG):
            dist = _masked_sqrt(accs[g])
            m0, m1 = masks[g]
            s0 = s0 + jnp.where(m0, dist, zero)
            c0 = c0 + jnp.where(m0, one, zero)
            s1 = s1 + jnp.where(m1, dist, zero)
            c1 = c1 + jnp.where(m1, one, zero)
        acc_bc[b] = [s0, c0, s1, c1]

    for b in range(B):
        for j in range(4):
            outv[b * 4 + j, :] = acc_bc[b][j]
    pltpu.sync_copy(outv, out.at[wid])


@functools.partial(jax.jit)
def _sc_call(feat, tgt, p0, p1):
    mesh = plsc.VectorSubcoreMesh(core_axis_name="c", subcore_axis_name="s")
    return pl.kernel(
        _sc_body,
        out_type=jax.ShapeDtypeStruct((NW, 2 * NCLS * B, L), jnp.float32),
        mesh=mesh,
        scratch_types=[
            pltpu.VMEM((C, WBLK), jnp.float32),
            pltpu.VMEM((C, WBLK), jnp.float32),
            pltpu.VMEM((B, PPW), jnp.int32),
            pltpu.VMEM((C, L), jnp.float32),
            pltpu.VMEM((C, L), jnp.float32),
            pltpu.VMEM((2 * NCLS * B, L), jnp.float32),
            pltpu.SemaphoreType.DMA,
            pltpu.SemaphoreType.DMA,
        ],
    )(feat, tgt, p0, p1)


def kernel(features, predictions, targets, prototypes):
    del predictions
    feat = features.reshape(B, C, N)
    tgt = targets.reshape(B, N)
    p0 = jnp.broadcast_to(prototypes[0][:, None], (C, L))
    p1 = jnp.broadcast_to(prototypes[1][:, None], (C, L))
    part = _sc_call(feat, tgt, p0, p1)
    part = part.reshape(NW, B, NCLS, 2, L)
    sums = part.sum(axis=(0, 4))
    s = sums[..., 0]
    n = sums[..., 1]
    mean = jnp.where(n > 0, s / jnp.maximum(n, 1.0), 0.0)
    total = mean.sum()
    valid = (n > 0).astype(jnp.float32).sum()
    return jnp.where(valid > 0, total / valid, jnp.float32(0.0))


# speedup vs baseline: 1.3075x; 1.0556x over previous
import functools

import jax
import jax.numpy as jnp
from jax import lax
from jax.experimental import pallas as pl
from jax.experimental.pallas import tpu as pltpu
from jax.experimental.pallas import tpu_sc as plsc

NC, NS, L = 2, 16, 16
NW = NC * NS
B = 2
C = 128
NCLS = 2
N = 32 * 32 * 32
PPW = N // NW
WBLK = 128
NBLK = PPW // WBLK
NSTEP = B * NBLK
NG = WBLK // L
KU = 4


def _masked_sqrt(x):
    xc = jnp.maximum(x, jnp.float32(1e-12))
    i = lax.bitcast_convert_type(xc, jnp.int32)
    seed = jnp.full((L,), 0x5F3759DF, jnp.int32) - (i >> 1)
    y = lax.bitcast_convert_type(seed, jnp.float32)
    for _ in range(3):
        y = y * (jnp.float32(1.5) - jnp.float32(0.5) * xc * y * y)
    return x * y


def _sc_body(feat, tgt, p0, p1, out, fb0, fb1, tgtv, p0v, p1v, outv, sem0, sem1):
    wid = lax.axis_index("s") * NC + lax.axis_index("c")
    base = wid * PPW
    pltpu.sync_copy(p0, p0v)
    pltpu.sync_copy(p1, p1v)
    for b in range(B):
        pltpu.sync_copy(tgt.at[b, pl.ds(base, PPW)], tgtv.at[b])
    zero = jnp.zeros((L,), jnp.float32)
    one = jnp.ones((L,), jnp.float32)
    for j in range(2 * NCLS * B):
        outv[j, :] = zero

    def start(step, buf, sem):
        b = step // NBLK
        blk = lax.rem(step, NBLK)
        pltpu.async_copy(
            feat.at[b, :, pl.ds(base + blk * WBLK, WBLK)], buf, sem)

    start(0, fb0, sem0)
    start(1, fb1, sem1)

    def process(step, buf, sem):
        b = step // NBLK
        blk = lax.rem(step, NBLK)
        pltpu.make_async_copy(feat.at[0, :, pl.ds(0, WBLK)], buf, sem).wait()

        masks = []
        for g in range(NG):
            t = tgtv[b, pl.ds(blk * WBLK + g * L, L)]
            masks.append((t == 0, t == 1))

        def kbody(i, carry):
            accs = list(carry)
            for kk in range(KU):
                k = i * KU + kk
                pk0 = p0v[k, :]
                pk1 = p1v[k, :]
                for g in range(NG):
                    v = buf[k, pl.ds(g * L, L)]
                    psel = jnp.where(masks[g][0], pk0, pk1)
                    d = v - psel
                    accs[g] = accs[g] + d * d
            return tuple(accs)

        accs = lax.fori_loop(0, C // KU, kbody, tuple(zero for _ in range(NG)),
                             unroll=1)

        @pl.when(step + 2 < NSTEP)
        def _():
            start(step + 2, buf, sem)

        s0 = zero
        c0 = zero
        s1 = zero
        c1 = zero
        for g in range(NG):
            dist = _masked_sqrt(accs[g])
            m0, m1 = masks[g]
            s0 = s0 + jnp.where(m0, dist, zero)
            c0 = c0 + jnp.where(m0, one, zero)
            s1 = s1 + jnp.where(m1, dist, zero)
            c1 = c1 + jnp.where(m1, one, zero)
        row = b * 4
        outv[row, :] = outv[row, :] + s0
        outv[row + 1, :] = outv[row + 1, :] + c0
        outv[row + 2, :] = outv[row + 2, :] + s1
        outv[row + 3, :] = outv[row + 3, :] + c1

    def loop_body(j, carry):
        process(2 * j, fb0, sem0)
        process(2 * j + 1, fb1, sem1)
        return carry

    lax.fori_loop(0, NSTEP // 2, loop_body, jnp.int32(0), unroll=1)
    pltpu.sync_copy(outv, out.at[wid])


@functools.partial(jax.jit)
def _sc_call(feat, tgt, p0, p1):
    mesh = plsc.VectorSubcoreMesh(core_axis_name="c", subcore_axis_name="s")
    return pl.kernel(
        _sc_body,
        out_type=jax.ShapeDtypeStruct((NW, 2 * NCLS * B, L), jnp.float32),
        mesh=mesh,
        scratch_types=[
            pltpu.VMEM((C, WBLK), jnp.float32),
            pltpu.VMEM((C, WBLK), jnp.float32),
            pltpu.VMEM((B, PPW), jnp.int32),
            pltpu.VMEM((C, L), jnp.float32),
            pltpu.VMEM((C, L), jnp.float32),
            pltpu.VMEM((2 * NCLS * B, L), jnp.float32),
            pltpu.SemaphoreType.DMA,
            pltpu.SemaphoreType.DMA,
        ],
    )(feat, tgt, p0, p1)


def kernel(features, predictions, targets, prototypes):
    del predictions
    feat = features.reshape(B, C, N)
    tgt = targets.reshape(B, N)
    p0 = jnp.broadcast_to(prototypes[0][:, None], (C, L))
    p1 = jnp.broadcast_to(prototypes[1][:, None], (C, L))
    part = _sc_call(feat, tgt, p0, p1)
    part = part.reshape(NW, B, NCLS, 2, L)
    sums = part.sum(axis=(0, 4))
    s = sums[..., 0]
    n = sums[..., 1]
    mean = jnp.where(n > 0, s / jnp.maximum(n, 1.0), 0.0)
    total = mean.sum()
    valid = (n > 0).astype(jnp.float32).sum()
    return jnp.where(valid > 0, total / valid, jnp.float32(0.0))


# speedup vs baseline: 1.6329x; 1.2489x over previous
import functools

import jax
import jax.numpy as jnp
from jax import lax
from jax.experimental import pallas as pl
from jax.experimental.pallas import tpu as pltpu
from jax.experimental.pallas import tpu_sc as plsc

NC, NS, L = 2, 16, 16
NW = NC * NS
B = 2
C = 128
CG = C // L
NCLS = 2
N = 32 * 32 * 32
PPW = N // NW
PBLK = 128
NBLK = PPW // PBLK
NSTEP = B * NBLK
GPB = PBLK // L


def _masked_sqrt(x):
    xc = jnp.maximum(x, jnp.float32(1e-12))
    i = lax.bitcast_convert_type(xc, jnp.int32)
    seed = jnp.full((L,), 0x5F3759DF, jnp.int32) - (i >> 1)
    y = lax.bitcast_convert_type(seed, jnp.float32)
    for _ in range(3):
        y = y * (jnp.float32(1.5) - jnp.float32(0.5) * xc * y * y)
    return x * y


def _sc_body(feat, tgt, protos, out, fb0, fb1, tgtv, pv, stg, outv, sem0, sem1):
    wid = lax.axis_index("s") * NC + lax.axis_index("c")
    base = wid * PPW
    pltpu.sync_copy(protos, pv)
    for b in range(B):
        pltpu.sync_copy(tgt.at[b, pl.ds(base, PPW)], tgtv.at[pl.ds(b * PPW, PPW)])
    zero = jnp.zeros((L,), jnp.float32)
    one = jnp.ones((L,), jnp.float32)
    for j in range(2 * NCLS * B):
        outv[j, :] = zero

    p0 = [pv[0, pl.ds(j * L, L)] for j in range(CG)]
    p1 = [pv[1, pl.ds(j * L, L)] for j in range(CG)]

    def start(step, buf, sem):
        b = step // NBLK
        blk = lax.rem(step, NBLK)
        pltpu.async_copy(
            feat.at[b, pl.ds(base + blk * PBLK, PBLK), :], buf, sem)

    start(0, fb0, sem0)
    start(1, fb1, sem1)

    def process(step, buf, sem):
        b = step // NBLK
        blk = lax.rem(step, NBLK)
        pltpu.make_async_copy(feat.at[0, pl.ds(0, PBLK), :], buf, sem).wait()

        def gbody(g, carry):
            s0, c0, s1, c1 = carry
            for p in range(L):
                pix = g * L + p
                tsp = plsc.load_gather(
                    tgtv, [jnp.full((L,), b * PPW + blk * PBLK + pix, jnp.int32)])
                m = tsp == 0
                acc_a = zero
                acc_b = zero
                for j in range(CG):
                    v = buf[pix, pl.ds(j * L, L)]
                    psel = jnp.where(m, p0[j], p1[j])
                    d = v - psel
                    if j % 2 == 0:
                        acc_a = acc_a + d * d
                    else:
                        acc_b = acc_b + d * d
                stg[pl.ds(p * L, L)] = acc_a + acc_b
            rowbase = lax.iota(jnp.int32, L) * L
            d2 = zero
            for j in range(L):
                d2 = d2 + plsc.load_gather(stg, [rowbase + j])
            tvec = tgtv[pl.ds(b * PPW + blk * PBLK + g * L, L)]
            m0 = tvec == 0
            m1 = tvec == 1
            dist = _masked_sqrt(d2)
            s0 = s0 + jnp.where(m0, dist, zero)
            c0 = c0 + jnp.where(m0, one, zero)
            s1 = s1 + jnp.where(m1, dist, zero)
            c1 = c1 + jnp.where(m1, one, zero)
            return (s0, c0, s1, c1)

        s0, c0, s1, c1 = lax.fori_loop(
            0, GPB, gbody, (zero, zero, zero, zero), unroll=1)

        @pl.when(step + 2 < NSTEP)
        def _():
            start(step + 2, buf, sem)

        row = b * 4
        outv[row, :] = outv[row, :] + s0
        outv[row + 1, :] = outv[row + 1, :] + c0
        outv[row + 2, :] = outv[row + 2, :] + s1
        outv[row + 3, :] = outv[row + 3, :] + c1

    def loop_body(j, carry):
        process(2 * j, fb0, sem0)
        process(2 * j + 1, fb1, sem1)
        return carry

    lax.fori_loop(0, NSTEP // 2, loop_body, jnp.int32(0), unroll=1)
    pltpu.sync_copy(outv, out.at[wid])


@functools.partial(jax.jit)
def _sc_call(feat, tgt, protos):
    mesh = plsc.VectorSubcoreMesh(core_axis_name="c", subcore_axis_name="s")
    return pl.kernel(
        _sc_body,
        out_type=jax.ShapeDtypeStruct((NW, 2 * NCLS * B, L), jnp.float32),
        mesh=mesh,
        compiler_params=pltpu.CompilerParams(needs_layout_passes=False),
        scratch_types=[
            pltpu.VMEM((PBLK, C), jnp.float32),
            pltpu.VMEM((PBLK, C), jnp.float32),
            pltpu.VMEM((B * PPW,), jnp.int32),
            pltpu.VMEM((NCLS, C), jnp.float32),
            pltpu.VMEM((L * L,), jnp.float32),
            pltpu.VMEM((2 * NCLS * B, L), jnp.float32),
            pltpu.SemaphoreType.DMA,
            pltpu.SemaphoreType.DMA,
        ],
    )(feat, tgt, protos)


def kernel(features, predictions, targets, prototypes):
    del predictions
    feat = jnp.transpose(features, (0, 2, 3, 4, 1)).reshape(B, N, C)
    tgt = targets.reshape(B, N)
    part = _sc_call(feat, tgt, prototypes)
    part = part.reshape(NW, B, NCLS, 2, L)
    sums = part.sum(axis=(0, 4))
    s = sums[..., 0]
    n = sums[..., 1]
    mean = jnp.where(n > 0, s / jnp.maximum(n, 1.0), 0.0)
    total = mean.sum()
    valid = (n > 0).astype(jnp.float32).sum()
    return jnp.where(valid > 0, total / valid, jnp.float32(0.0))


# speedup vs baseline: 1.6503x; 1.0106x over previous
import functools

import jax
import jax.numpy as jnp
from jax import lax
from jax.experimental import pallas as pl
from jax.experimental.pallas import tpu as pltpu
from jax.experimental.pallas import tpu_sc as plsc

NC, NS, L = 2, 16, 16
NW = NC * NS
B = 2
C = 128
CG = C // L
NCLS = 2
N = 32 * 32 * 32
PPW = N // NW
PBLK = 128
NBLK = PPW // PBLK
NSTEP = B * NBLK
GPB = PBLK // L


def _masked_sqrt(x):
    xc = jnp.maximum(x, jnp.float32(1e-12))
    i = lax.bitcast_convert_type(xc, jnp.int32)
    seed = jnp.full((L,), 0x5F3759DF, jnp.int32) - (i >> 1)
    y = lax.bitcast_convert_type(seed, jnp.float32)
    for _ in range(3):
        y = y * (jnp.float32(1.5) - jnp.float32(0.5) * xc * y * y)
    return x * y


def _sc_body(feat, tgt, protos, out, fb0, fb1, tgtv, pv, stg, outv, sem0, sem1):
    wid = lax.axis_index("s") * NC + lax.axis_index("c")
    base = wid * PPW
    pltpu.sync_copy(protos, pv)
    for b in range(B):
        pltpu.sync_copy(tgt.at[b, pl.ds(base, PPW)], tgtv.at[pl.ds(b * PPW, PPW)])
    zero = jnp.zeros((L,), jnp.float32)
    one = jnp.ones((L,), jnp.float32)
    for j in range(2 * NCLS * B):
        outv[j, :] = zero

    p0 = [pv[0, pl.ds(j * L, L)] for j in range(CG)]
    p1 = [pv[1, pl.ds(j * L, L)] for j in range(CG)]

    def start(step, buf, sem):
        b = step // NBLK
        blk = lax.rem(step, NBLK)
        pltpu.async_copy(
            feat.at[b, pl.ds(base + blk * PBLK, PBLK), :], buf, sem)

    start(0, fb0, sem0)
    start(1, fb1, sem1)

    def process(step, buf, sem):
        b = step // NBLK
        blk = lax.rem(step, NBLK)
        pltpu.make_async_copy(feat.at[0, pl.ds(0, PBLK), :], buf, sem).wait()

        def gbody(g, carry):
            s0, c0, s1, c1 = carry
            tvec = tgtv[pl.ds(b * PPW + blk * PBLK + g * L, L)]
            for p in range(L):
                pix = g * L + p
                tsp = jnp.take_along_axis(
                    tvec, jnp.full((L,), p, jnp.int32), axis=0)
                m = tsp == 0
                acc_a = zero
                acc_b = zero
                for j in range(CG):
                    v = buf[pix, pl.ds(j * L, L)]
                    psel = jnp.where(m, p0[j], p1[j])
                    d = v - psel
                    if j % 2 == 0:
                        acc_a = acc_a + d * d
                    else:
                        acc_b = acc_b + d * d
                stg[pl.ds(p * L, L)] = acc_a + acc_b
            rowbase = lax.iota(jnp.int32, L) * L
            d2 = zero
            for j in range(L):
                d2 = d2 + plsc.load_gather(stg, [rowbase + j])
            m0 = tvec == 0
            m1 = tvec == 1
            dist = _masked_sqrt(d2)
            s0 = s0 + jnp.where(m0, dist, zero)
            c0 = c0 + jnp.where(m0, one, zero)
            s1 = s1 + jnp.where(m1, dist, zero)
            c1 = c1 + jnp.where(m1, one, zero)
            return (s0, c0, s1, c1)

        s0, c0, s1, c1 = lax.fori_loop(
            0, GPB, gbody, (zero, zero, zero, zero), unroll=1)

        @pl.when(step + 2 < NSTEP)
        def _():
            start(step + 2, buf, sem)

        row = b * 4
        outv[row, :] = outv[row, :] + s0
        outv[row + 1, :] = outv[row + 1, :] + c0
        outv[row + 2, :] = outv[row + 2, :] + s1
        outv[row + 3, :] = outv[row + 3, :] + c1

    def loop_body(j, carry):
        process(2 * j, fb0, sem0)
        process(2 * j + 1, fb1, sem1)
        return carry

    lax.fori_loop(0, NSTEP // 2, loop_body, jnp.int32(0), unroll=1)
    pltpu.sync_copy(outv, out.at[wid])


@functools.partial(jax.jit)
def _sc_call(feat, tgt, protos):
    mesh = plsc.VectorSubcoreMesh(core_axis_name="c", subcore_axis_name="s")
    return pl.kernel(
        _sc_body,
        out_type=jax.ShapeDtypeStruct((NW, 2 * NCLS * B, L), jnp.float32),
        mesh=mesh,
        compiler_params=pltpu.CompilerParams(needs_layout_passes=False),
        scratch_types=[
            pltpu.VMEM((PBLK, C), jnp.float32),
            pltpu.VMEM((PBLK, C), jnp.float32),
            pltpu.VMEM((B * PPW,), jnp.int32),
            pltpu.VMEM((NCLS, C), jnp.float32),
            pltpu.VMEM((L * L,), jnp.float32),
            pltpu.VMEM((2 * NCLS * B, L), jnp.float32),
            pltpu.SemaphoreType.DMA,
            pltpu.SemaphoreType.DMA,
        ],
    )(feat, tgt, protos)


def kernel(features, predictions, targets, prototypes):
    del predictions
    feat = jnp.transpose(features, (0, 2, 3, 4, 1)).reshape(B, N, C)
    tgt = targets.reshape(B, N)
    part = _sc_call(feat, tgt, prototypes)
    part = part.reshape(NW, B, NCLS, 2, L)
    sums = part.sum(axis=(0, 4))
    s = sums[..., 0]
    n = sums[..., 1]
    mean = jnp.where(n > 0, s / jnp.maximum(n, 1.0), 0.0)
    total = mean.sum()
    valid = (n > 0).astype(jnp.float32).sum()
    return jnp.where(valid > 0, total / valid, jnp.float32(0.0))
